# baseline (speedup 1.0000x reference)
import jax
import jax.numpy as jnp
from jax.experimental import pallas as pl
from jax.experimental.pallas import tpu as pltpu


def _mlp_fused(x_ref, w1_ref, b1_ref, wp_ref, bp_ref, out_ref):
    feats = jnp.tanh(
        jnp.dot(x_ref[...], w1_ref[...], preferred_element_type=jnp.float32)
        + b1_ref[...]
    )
    out_ref[...] = (
        jnp.dot(feats, wp_ref[...], preferred_element_type=jnp.float32)
        + bp_ref[...]
    )


def kernel(inputs, W1, b1, W2, b2, Wp, bp):
    B, D = inputs.shape
    H = W1.shape[1]
    O = Wp.shape[1]
    bm = 1024
    xb = inputs.astype(jnp.bfloat16)
    w1b = W1.astype(jnp.bfloat16)
    b1r = b1.reshape(1, H)
    bpr = bp.reshape(1, O)
    out = pl.pallas_call(
        _mlp_fused,
        grid=(B // bm,),
        in_specs=[
            pl.BlockSpec((bm, D), lambda i: (i, 0)),
            pl.BlockSpec((D, H), lambda i: (0, 0)),
            pl.BlockSpec((1, H), lambda i: (0, 0)),
            pl.BlockSpec((D, O), lambda i: (0, 0)),
            pl.BlockSpec((1, O), lambda i: (0, 0)),
        ],
        out_specs=pl.BlockSpec((bm, O), lambda i: (i, 0)),
        out_shape=jax.ShapeDtypeStruct((B, O), jnp.float32),
        compiler_params=pltpu.CompilerParams(
            dimension_semantics=("parallel",),
        ),
    )(xb, w1b, b1r, Wp, bpr)
    return out


# speedup vs baseline: 1.1529x; 1.1529x over previous
import jax
import jax.numpy as jnp
from jax.experimental import pallas as pl
from jax.experimental.pallas import tpu as pltpu


def _mlp_fused(x_ref, w1_ref, b1_ref, wp_ref, bp_ref, out_ref):
    feats = jnp.tanh(
        jnp.dot(x_ref[...], w1_ref[...], preferred_element_type=jnp.float32)
        + b1_ref[...]
    )
    out_ref[...] = (
        jnp.dot(feats, wp_ref[...], preferred_element_type=jnp.float32)
        + bp_ref[...]
    )


def kernel(inputs, W1, b1, W2, b2, Wp, bp):
    B, D = inputs.shape
    H = W1.shape[1]
    O = Wp.shape[1]
    bm = 512
    xb = inputs
    w1b = W1
    b1r = b1.reshape(1, H)
    bpr = bp.reshape(1, O)
    out = pl.pallas_call(
        _mlp_fused,
        grid=(B // bm,),
        in_specs=[
            pl.BlockSpec((bm, D), lambda i: (i, 0)),
            pl.BlockSpec((D, H), lambda i: (0, 0)),
            pl.BlockSpec((1, H), lambda i: (0, 0)),
            pl.BlockSpec((D, O), lambda i: (0, 0)),
            pl.BlockSpec((1, O), lambda i: (0, 0)),
        ],
        out_specs=pl.BlockSpec((bm, O), lambda i: (i, 0)),
        out_shape=jax.ShapeDtypeStruct((B, O), jnp.float32),
        compiler_params=pltpu.CompilerParams(
            dimension_semantics=("parallel",),
        ),
    )(xb, w1b, b1r, Wp, bpr)
    return out


# speedup vs baseline: 1.7349x; 1.5048x over previous
import jax
import jax.numpy as jnp
from jax.experimental import pallas as pl
from jax.experimental.pallas import tpu as pltpu


def _mlp_fused(x_ref, w1_ref, b1_ref, wp_ref, bp_ref, out_ref):
    feats = jnp.tanh(
        jnp.dot(x_ref[...], w1_ref[...], preferred_element_type=jnp.float32)
        + b1_ref[...]
    )
    out_ref[...] = (
        jnp.dot(feats, wp_ref[...], preferred_element_type=jnp.float32)
        + bp_ref[...]
    )


def kernel(inputs, W1, b1, W2, b2, Wp, bp):
    B, D = inputs.shape
    H = W1.shape[1]
    O = Wp.shape[1]
    bm = 2048
    xb = inputs
    w1b = W1
    b1r = b1.reshape(1, H)
    bpr = bp.reshape(1, O)
    out = pl.pallas_call(
        _mlp_fused,
        grid=(B // bm,),
        in_specs=[
            pl.BlockSpec((bm, D), lambda i: (i, 0)),
            pl.BlockSpec((D, H), lambda i: (0, 0)),
            pl.BlockSpec((1, H), lambda i: (0, 0)),
            pl.BlockSpec((D, O), lambda i: (0, 0)),
            pl.BlockSpec((1, O), lambda i: (0, 0)),
        ],
        out_specs=pl.BlockSpec((bm, O), lambda i: (i, 0)),
        out_shape=jax.ShapeDtypeStruct((B, O), jnp.float32),
        compiler_params=pltpu.CompilerParams(
            dimension_semantics=("parallel",),
        ),
    )(xb, w1b, b1r, Wp, bpr)
    return out


# speedup vs baseline: 1.8061x; 1.0411x over previous
import jax
import jax.numpy as jnp
from jax.experimental import pallas as pl
from jax.experimental.pallas import tpu as pltpu


def _mlp_fused(x_ref, w1_ref, b1_ref, wp_ref, bp_ref, out_ref):
    feats = jnp.tanh(
        jnp.dot(x_ref[...], w1_ref[...], preferred_element_type=jnp.float32)
        + b1_ref[...]
    )
    out_ref[...] = (
        jnp.dot(feats, wp_ref[...], preferred_element_type=jnp.float32)
        + bp_ref[...]
    )


def kernel(inputs, W1, b1, W2, b2, Wp, bp):
    B, D = inputs.shape
    H = W1.shape[1]
    O = Wp.shape[1]
    bm = 4096
    xb = inputs
    w1b = W1
    b1r = b1.reshape(1, H)
    bpr = bp.reshape(1, O)
    out = pl.pallas_call(
        _mlp_fused,
        grid=(B // bm,),
        in_specs=[
            pl.BlockSpec((bm, D), lambda i: (i, 0)),
            pl.BlockSpec((D, H), lambda i: (0, 0)),
            pl.BlockSpec((1, H), lambda i: (0, 0)),
            pl.BlockSpec((D, O), lambda i: (0, 0)),
            pl.BlockSpec((1, O), lambda i: (0, 0)),
        ],
        out_specs=pl.BlockSpec((bm, O), lambda i: (i, 0)),
        out_shape=jax.ShapeDtypeStruct((B, O), jnp.float32),
        compiler_params=pltpu.CompilerParams(
            dimension_semantics=("parallel",),
        ),
    )(xb, w1b, b1r, Wp, bpr)
    return out
